# Initial kernel scaffold; baseline (speedup 1.0000x reference)
import functools

import jax
import jax.numpy as jnp
from jax import lax
from jax.experimental import pallas as pl
from jax.experimental.pallas import tpu as pltpu
from jax.experimental.pallas import tpu_sc as plsc

N_NODES = 10000
N_EDGES = 320000
IN_CH = 128
HID = 256

NC = 2
NS = 16
NW = NC * NS
BLK = 128
J = 80
E_PAD = NW * J * BLK
ROWS_ACC = 10240
RPT = ROWS_ACC // NS
W_AUG = 144

_F32 = jnp.float32
_HIGH = lax.Precision.HIGHEST


@functools.cache
def _make_sc_agg(width):
    mesh = plsc.VectorSubcoreMesh(core_axis_name="c", subcore_axis_name="s")

    @functools.partial(
        pl.kernel,
        out_type=jax.ShapeDtypeStruct((NC, ROWS_ACC, width), _F32),
        mesh=mesh,
        scratch_types=[
            pltpu.VMEM((J, BLK), jnp.int32),
            pltpu.VMEM((J, BLK), jnp.int32),
            pltpu.VMEM((2, BLK, width), _F32),
            pltpu.MemorySpace.VMEM_SHARED((ROWS_ACC, width), _F32),
            pltpu.SemaphoreType.DMA,
            pltpu.SemaphoreType.DMA,
        ],
    )
    def sc_agg(table, srcb, dstb, zrow, out, src_v, dst_v, buf, acc, g0, g1):
        c = lax.axis_index("c")
        s = lax.axis_index("s")
        wid = s * NC + c
        pltpu.sync_copy(srcb.at[wid], src_v)
        pltpu.sync_copy(dstb.at[wid], dst_v)
        sems = (g0, g1)

        def gather(j, slot):
            pltpu.async_copy(table.at[src_v.at[j]], buf.at[slot], sems[slot])

        gather(0, 0)
        pltpu.sync_copy(zrow, acc.at[pl.ds(s * RPT, RPT)])
        plsc.subcore_barrier()

        def step(i, carry):
            j0 = i * 2
            for u in range(2):
                j = j0 + u

                @pl.when(j + 1 < J)
                def _():
                    gather(j + 1, 1 - u)

                pltpu.make_async_copy(
                    table.at[src_v.at[j]], buf.at[u], sems[u]
                ).wait()
                pltpu.sync_copy(buf.at[u], acc.at[dst_v.at[j]], add=True)
            return carry

        lax.fori_loop(0, J // 2, step, 0)
        plsc.subcore_barrier()
        pltpu.sync_copy(
            acc.at[pl.ds(s * RPT, RPT)], out.at[c, pl.ds(s * RPT, RPT)]
        )

    return sc_agg


def _dot(a, b):
    return jnp.dot(a, b, precision=_HIGH, preferred_element_type=_F32)


_R = 1000


def _tc1_body(acc_ref, x_ref, wl1_ref, wr1_ref, b1_ref, wl2_ref,
              h1_ref, p2_ref, inv_ref):
    a = acc_ref[0] + acc_ref[1]
    deg = a[:, IN_CH:IN_CH + 1]
    inv = 1.0 / jnp.maximum(deg, 1.0)
    aggn = a[:, :IN_CH] * inv
    h1 = jnp.maximum(
        _dot(aggn, wl1_ref[...]) + b1_ref[...] + _dot(x_ref[...], wr1_ref[...]),
        0.0,
    )
    h1_ref[...] = h1
    p2_ref[...] = _dot(h1, wl2_ref[...])
    inv_ref[...] = inv


def _tc1(acc, x, Wl1, Wr1, b1, Wl2):
    full = lambda i: (0, 0)
    return pl.pallas_call(
        _tc1_body,
        grid=(N_NODES // _R,),
        in_specs=[
            pl.BlockSpec((NC, _R, W_AUG), lambda i: (0, i, 0)),
            pl.BlockSpec((_R, IN_CH), lambda i: (i, 0)),
            pl.BlockSpec((IN_CH, HID), full),
            pl.BlockSpec((IN_CH, HID), full),
            pl.BlockSpec((1, HID), full),
            pl.BlockSpec((HID, 32), full),
        ],
        out_specs=[
            pl.BlockSpec((_R, HID), lambda i: (i, 0)),
            pl.BlockSpec((_R, 32), lambda i: (i, 0)),
            pl.BlockSpec((_R, 1), lambda i: (i, 0)),
        ],
        out_shape=[
            jax.ShapeDtypeStruct((N_NODES, HID), _F32),
            jax.ShapeDtypeStruct((N_NODES, 32), _F32),
            jax.ShapeDtypeStruct((N_NODES, 1), _F32),
        ],
    )(acc, x, Wl1, Wr1, b1.reshape(1, HID), Wl2)


def _tc2_body(acc_ref, inv_ref, h1_ref, wr2_ref, b2_ref, wl3_ref,
              h2_ref, p3_ref):
    a = (acc_ref[0] + acc_ref[1]) * inv_ref[...]
    h2 = jnp.maximum(a + b2_ref[...] + _dot(h1_ref[...], wr2_ref[...]), 0.0)
    h2_ref[...] = h2
    p3_ref[...] = _dot(h2, wl3_ref[...])


def _tc2(acc, inv, h1, Wr2, b2, Wl3):
    full = lambda i: (0, 0)
    return pl.pallas_call(
        _tc2_body,
        grid=(N_NODES // _R,),
        in_specs=[
            pl.BlockSpec((NC, _R, 32), lambda i: (0, i, 0)),
            pl.BlockSpec((_R, 1), lambda i: (i, 0)),
            pl.BlockSpec((_R, HID), lambda i: (i, 0)),
            pl.BlockSpec((HID, 32), full),
            pl.BlockSpec((1, 32), full),
            pl.BlockSpec((32, 16), full),
        ],
        out_specs=[
            pl.BlockSpec((_R, 32), lambda i: (i, 0)),
            pl.BlockSpec((_R, 16), lambda i: (i, 0)),
        ],
        out_shape=[
            jax.ShapeDtypeStruct((N_NODES, 32), _F32),
            jax.ShapeDtypeStruct((N_NODES, 16), _F32),
        ],
    )(acc, inv, h1, Wr2, b2.reshape(1, 32), Wl3)


def _tc3_body(acc_ref, inv_ref, h2_ref, wr3_ref, b3_ref, fw1_ref, fb1_ref,
              fw2_ref, fb2_ref, out_ref):
    a = (acc_ref[0] + acc_ref[1]) * inv_ref[...]
    h3 = jnp.maximum(a + b3_ref[...] + _dot(h2_ref[...], wr3_ref[...]), 0.0)
    t = jnp.maximum(_dot(h3, fw1_ref[...]) + fb1_ref[...], 0.0)
    o = _dot(t, fw2_ref[...]) + fb2_ref[...]
    m = jnp.max(o, axis=1, keepdims=True)
    sh = o - m
    out_ref[...] = sh - jnp.log(jnp.sum(jnp.exp(sh), axis=1, keepdims=True))


def _tc3(acc, inv, h2, Wr3, b3, fcW1, fcb1, fcW2, fcb2):
    full = lambda i: (0, 0)
    return pl.pallas_call(
        _tc3_body,
        grid=(N_NODES // _R,),
        in_specs=[
            pl.BlockSpec((NC, _R, 16), lambda i: (0, i, 0)),
            pl.BlockSpec((_R, 1), lambda i: (i, 0)),
            pl.BlockSpec((_R, 32), lambda i: (i, 0)),
            pl.BlockSpec((32, 16), full),
            pl.BlockSpec((1, 16), full),
            pl.BlockSpec((16, 8), full),
            pl.BlockSpec((1, 8), full),
            pl.BlockSpec((8, 2), full),
            pl.BlockSpec((1, 2), full),
        ],
        out_specs=pl.BlockSpec((_R, 2), lambda i: (i, 0)),
        out_shape=jax.ShapeDtypeStruct((N_NODES, 2), _F32),
    )(acc, inv, h2, Wr3, b3.reshape(1, 16), fcW1, fcb1.reshape(1, 8),
      fcW2, fcb2.reshape(1, 2))


def kernel(x, edge_index, Wl1, Wr1, b1, Wl2, Wr2, b2, Wl3, Wr3, b3,
           fcW1, fcb1, fcW2, fcb2):
    src = edge_index[0].astype(jnp.int32)
    dst = edge_index[1].astype(jnp.int32)
    pad = E_PAD - N_EDGES
    src_r = jnp.concatenate(
        [src, jnp.zeros((pad,), jnp.int32)]).reshape(NW, J, BLK)
    dst_r = jnp.concatenate(
        [dst, jnp.full((pad,), N_NODES, jnp.int32)]).reshape(NW, J, BLK)

    x_aug = jnp.concatenate(
        [x, jnp.ones((N_NODES, 1), _F32), jnp.zeros((N_NODES, 15), _F32)],
        axis=1)

    acc1 = _make_sc_agg(W_AUG)(
        x_aug, src_r, dst_r, jnp.zeros((RPT, W_AUG), _F32))
    h1, p2, inv = _tc1(acc1, x, Wl1, Wr1, b1, Wl2)
    acc2 = _make_sc_agg(32)(p2, src_r, dst_r, jnp.zeros((RPT, 32), _F32))
    h2, p3 = _tc2(acc2, inv, h1, Wr2, b2, Wl3)
    acc3 = _make_sc_agg(16)(p3, src_r, dst_r, jnp.zeros((RPT, 16), _F32))
    return _tc3(acc3, inv, h2, Wr3, b3, fcW1, fcb1, fcW2, fcb2)


# speedup vs baseline: 9.9087x; 9.9087x over previous
import functools

import jax
import jax.numpy as jnp
from jax import lax
from jax.experimental import pallas as pl
from jax.experimental.pallas import tpu as pltpu
from jax.experimental.pallas import tpu_sc as plsc

N_NODES = 10000
N_EDGES = 320000
IN_CH = 128
HID = 256

NC = 2
NS = 16
NW = NC * NS
BLK = 80
J = 126
E_PAD = NW * J * BLK
ROWS_ACC = 10240
RPT = ROWS_ACC // NS
W_AUG = 136

_F32 = jnp.float32
_HIGH = lax.Precision.HIGHEST


@functools.cache
def _make_sc_agg(width):
    mesh = plsc.VectorSubcoreMesh(core_axis_name="c", subcore_axis_name="s")

    @functools.partial(
        pl.kernel,
        out_type=jax.ShapeDtypeStruct((NC, ROWS_ACC, width), _F32),
        mesh=mesh,
        scratch_types=[
            pltpu.VMEM((J, BLK), jnp.int32),
            pltpu.VMEM((J, BLK), jnp.int32),
            pltpu.VMEM((2, BLK, width), _F32),
            pltpu.MemorySpace.VMEM_SHARED((ROWS_ACC, width), _F32),
            pltpu.SemaphoreType.DMA,
            pltpu.SemaphoreType.DMA,
        ],
        compiler_params=pltpu.CompilerParams(use_tc_tiling_on_sc=False),
    )
    def sc_agg(table, srcb, dstb, zrow, out, src_v, dst_v, buf, acc, g0, g1):
        c = lax.axis_index("c")
        s = lax.axis_index("s")
        wid = s * NC + c
        pltpu.sync_copy(srcb.at[wid], src_v)
        pltpu.sync_copy(dstb.at[wid], dst_v)
        sems = (g0, g1)

        def gather(j, slot):
            pltpu.async_copy(table.at[src_v.at[j]], buf.at[slot], sems[slot])

        gather(0, 0)
        pltpu.sync_copy(zrow, acc.at[pl.ds(s * RPT, RPT)])
        plsc.subcore_barrier()

        def step(i, carry):
            j0 = i * 2
            for u in range(2):
                j = j0 + u

                @pl.when(j + 1 < J)
                def _():
                    gather(j + 1, 1 - u)

                pltpu.make_async_copy(
                    table.at[src_v.at[j]], buf.at[u], sems[u]
                ).wait()
                pltpu.sync_copy(buf.at[u], acc.at[dst_v.at[j]], add=True)
            return carry

        lax.fori_loop(0, J // 2, step, 0)
        plsc.subcore_barrier()
        pltpu.sync_copy(
            acc.at[pl.ds(s * RPT, RPT)], out.at[c, pl.ds(s * RPT, RPT)]
        )

    return sc_agg


def _dot(a, b):
    return jnp.dot(a, b, precision=_HIGH, preferred_element_type=_F32)


_R = 1000


def _tc1_body(acc_ref, x_ref, wl1_ref, wr1_ref, b1_ref, wl2_ref,
              h1_ref, p2_ref, inv_ref):
    a = acc_ref[0] + acc_ref[1]
    deg = a[:, IN_CH:IN_CH + 1]
    inv = 1.0 / jnp.maximum(deg, 1.0)
    aggn = a[:, :IN_CH] * inv
    h1 = jnp.maximum(
        _dot(aggn, wl1_ref[...]) + b1_ref[...] + _dot(x_ref[...], wr1_ref[...]),
        0.0,
    )
    h1_ref[...] = h1
    p2_ref[...] = _dot(h1, wl2_ref[...])
    inv_ref[...] = inv


def _tc1(acc, x, Wl1, Wr1, b1, Wl2):
    full = lambda i: (0, 0)
    return pl.pallas_call(
        _tc1_body,
        grid=(N_NODES // _R,),
        in_specs=[
            pl.BlockSpec((NC, _R, W_AUG), lambda i: (0, i, 0)),
            pl.BlockSpec((_R, IN_CH), lambda i: (i, 0)),
            pl.BlockSpec((IN_CH, HID), full),
            pl.BlockSpec((IN_CH, HID), full),
            pl.BlockSpec((1, HID), full),
            pl.BlockSpec((HID, 32), full),
        ],
        out_specs=[
            pl.BlockSpec((_R, HID), lambda i: (i, 0)),
            pl.BlockSpec((_R, 32), lambda i: (i, 0)),
            pl.BlockSpec((_R, 1), lambda i: (i, 0)),
        ],
        out_shape=[
            jax.ShapeDtypeStruct((N_NODES, HID), _F32),
            jax.ShapeDtypeStruct((N_NODES, 32), _F32),
            jax.ShapeDtypeStruct((N_NODES, 1), _F32),
        ],
    )(acc, x, Wl1, Wr1, b1.reshape(1, HID), Wl2)


def _tc2_body(acc_ref, inv_ref, h1_ref, wr2_ref, b2_ref, wl3_ref,
              h2_ref, p3_ref):
    a = (acc_ref[0] + acc_ref[1]) * inv_ref[...]
    h2 = jnp.maximum(a + b2_ref[...] + _dot(h1_ref[...], wr2_ref[...]), 0.0)
    h2_ref[...] = h2
    p3_ref[...] = _dot(h2, wl3_ref[...])


def _tc2(acc, inv, h1, Wr2, b2, Wl3):
    full = lambda i: (0, 0)
    return pl.pallas_call(
        _tc2_body,
        grid=(N_NODES // _R,),
        in_specs=[
            pl.BlockSpec((NC, _R, 32), lambda i: (0, i, 0)),
            pl.BlockSpec((_R, 1), lambda i: (i, 0)),
            pl.BlockSpec((_R, HID), lambda i: (i, 0)),
            pl.BlockSpec((HID, 32), full),
            pl.BlockSpec((1, 32), full),
            pl.BlockSpec((32, 16), full),
        ],
        out_specs=[
            pl.BlockSpec((_R, 32), lambda i: (i, 0)),
            pl.BlockSpec((_R, 16), lambda i: (i, 0)),
        ],
        out_shape=[
            jax.ShapeDtypeStruct((N_NODES, 32), _F32),
            jax.ShapeDtypeStruct((N_NODES, 16), _F32),
        ],
    )(acc, inv, h1, Wr2, b2.reshape(1, 32), Wl3)


def _tc3_body(acc_ref, inv_ref, h2_ref, wr3_ref, b3_ref, fw1_ref, fb1_ref,
              fw2_ref, fb2_ref, out_ref):
    a = (acc_ref[0] + acc_ref[1]) * inv_ref[...]
    h3 = jnp.maximum(a + b3_ref[...] + _dot(h2_ref[...], wr3_ref[...]), 0.0)
    t = jnp.maximum(_dot(h3, fw1_ref[...]) + fb1_ref[...], 0.0)
    o = _dot(t, fw2_ref[...]) + fb2_ref[...]
    m = jnp.max(o, axis=1, keepdims=True)
    sh = o - m
    out_ref[...] = sh - jnp.log(jnp.sum(jnp.exp(sh), axis=1, keepdims=True))


def _tc3(acc, inv, h2, Wr3, b3, fcW1, fcb1, fcW2, fcb2):
    full = lambda i: (0, 0)
    return pl.pallas_call(
        _tc3_body,
        grid=(N_NODES // _R,),
        in_specs=[
            pl.BlockSpec((NC, _R, 16), lambda i: (0, i, 0)),
            pl.BlockSpec((_R, 1), lambda i: (i, 0)),
            pl.BlockSpec((_R, 32), lambda i: (i, 0)),
            pl.BlockSpec((32, 16), full),
            pl.BlockSpec((1, 16), full),
            pl.BlockSpec((16, 8), full),
            pl.BlockSpec((1, 8), full),
            pl.BlockSpec((8, 2), full),
            pl.BlockSpec((1, 2), full),
        ],
        out_specs=pl.BlockSpec((_R, 2), lambda i: (i, 0)),
        out_shape=jax.ShapeDtypeStruct((N_NODES, 2), _F32),
    )(acc, inv, h2, Wr3, b3.reshape(1, 16), fcW1, fcb1.reshape(1, 8),
      fcW2, fcb2.reshape(1, 2))


def kernel(x, edge_index, Wl1, Wr1, b1, Wl2, Wr2, b2, Wl3, Wr3, b3,
           fcW1, fcb1, fcW2, fcb2):
    src = edge_index[0].astype(jnp.int32)
    dst = edge_index[1].astype(jnp.int32)
    pad = E_PAD - N_EDGES
    src_r = jnp.concatenate(
        [src, jnp.zeros((pad,), jnp.int32)]).reshape(NW, J, BLK)
    dst_r = jnp.concatenate(
        [dst, jnp.full((pad,), N_NODES, jnp.int32)]).reshape(NW, J, BLK)

    x_aug = jnp.concatenate(
        [x, jnp.ones((N_NODES, 1), _F32), jnp.zeros((N_NODES, 7), _F32)],
        axis=1)

    acc1 = _make_sc_agg(W_AUG)(
        x_aug, src_r, dst_r, jnp.zeros((RPT, W_AUG), _F32))
    h1, p2, inv = _tc1(acc1, x, Wl1, Wr1, b1, Wl2)
    acc2 = _make_sc_agg(32)(p2, src_r, dst_r, jnp.zeros((RPT, 32), _F32))
    h2, p3 = _tc2(acc2, inv, h1, Wr2, b2, Wl3)
    acc3 = _make_sc_agg(16)(p3, src_r, dst_r, jnp.zeros((RPT, 16), _F32))
    return _tc3(acc3, inv, h2, Wr3, b3, fcW1, fcb1, fcW2, fcb2)


# speedup vs baseline: 16.9223x; 1.7078x over previous
import functools

import jax
import jax.numpy as jnp
from jax import lax
from jax.experimental import pallas as pl
from jax.experimental.pallas import tpu as pltpu
from jax.experimental.pallas import tpu_sc as plsc

N_NODES = 10000
N_EDGES = 320000
IN_CH = 128
HID = 256

NC = 2
NS = 16
NW = NC * NS
BLK = 80
J = N_EDGES // (NW * BLK)
ROWS_ACC = 10240
RPT = ROWS_ACC // NS
DEG_W = 8

_F32 = jnp.float32


@functools.cache
def _make_sc_agg(width, with_deg):
    mesh = plsc.VectorSubcoreMesh(core_axis_name="c", subcore_axis_name="s")

    out_type = [jax.ShapeDtypeStruct((NC, ROWS_ACC, width), _F32)]
    scratch = [
        pltpu.VMEM((J, BLK), jnp.int32),
        pltpu.VMEM((J, BLK), jnp.int32),
        pltpu.VMEM((2, BLK, width), _F32),
        pltpu.MemorySpace.VMEM_SHARED((ROWS_ACC, width), _F32),
        pltpu.SemaphoreType.DMA,
        pltpu.SemaphoreType.DMA,
    ]
    if with_deg:
        out_type.append(jax.ShapeDtypeStruct((NC, ROWS_ACC, DEG_W), _F32))
        scratch += [
            pltpu.VMEM((BLK, DEG_W), _F32),
            pltpu.MemorySpace.VMEM_SHARED((ROWS_ACC, DEG_W), _F32),
        ]

    @functools.partial(
        pl.kernel,
        out_type=out_type,
        mesh=mesh,
        scratch_types=scratch,
        compiler_params=pltpu.CompilerParams(use_tc_tiling_on_sc=False),
    )
    def sc_agg(table, edges, zrow, *rest):
        if with_deg:
            zdeg, ones, out, outd, src_v, dst_v, buf, acc, g0, g1, ones_v, accd = rest
        else:
            out, src_v, dst_v, buf, acc, g0, g1 = rest
        c = lax.axis_index("c")
        s = lax.axis_index("s")
        wid = s * NC + c
        pltpu.sync_copy(edges.at[0, wid], src_v)
        pltpu.sync_copy(edges.at[1, wid], dst_v)
        sems = (g0, g1)

        def gather(j, slot):
            pltpu.async_copy(table.at[src_v.at[j]], buf.at[slot], sems[slot])

        gather(0, 0)
        pltpu.sync_copy(zrow, acc.at[pl.ds(s * RPT, RPT)])
        if with_deg:
            pltpu.sync_copy(ones, ones_v)
            pltpu.sync_copy(zdeg, accd.at[pl.ds(s * RPT, RPT)])
        plsc.subcore_barrier()

        def do_j(j, u):
            @pl.when(j + 1 < J)
            def _():
                gather(j + 1, 1 - u)

            pltpu.make_async_copy(
                table.at[src_v.at[j]], buf.at[u], sems[u]
            ).wait()
            pltpu.sync_copy(buf.at[u], acc.at[dst_v.at[j]], add=True)
            if with_deg:
                pltpu.sync_copy(ones_v, accd.at[dst_v.at[j]], add=True)

        def step(i, carry):
            for u in range(2):
                do_j(i * 2 + u, u)
            return carry

        lax.fori_loop(0, J // 2, step, 0)
        do_j(J - 1, 0)
        plsc.subcore_barrier()
        pltpu.sync_copy(
            acc.at[pl.ds(s * RPT, RPT)], out.at[c, pl.ds(s * RPT, RPT)]
        )
        if with_deg:
            pltpu.sync_copy(
                accd.at[pl.ds(s * RPT, RPT)], outd.at[c, pl.ds(s * RPT, RPT)]
            )

    return sc_agg


def _dot(a, b):
    return jnp.dot(a, b, preferred_element_type=_F32)


_R = 2000


def _tc1_body(acc_ref, accd_ref, x_ref, wl1_ref, wr1_ref, b1_ref, wl2_ref,
              h1_ref, p2_ref, inv_ref):
    deg = accd_ref[0, :, :1] + accd_ref[1, :, :1]
    inv = 1.0 / jnp.maximum(deg, 1.0)
    aggn = (acc_ref[0] + acc_ref[1]) * inv
    h1 = jnp.maximum(
        _dot(aggn, wl1_ref[...]) + b1_ref[...] + _dot(x_ref[...], wr1_ref[...]),
        0.0,
    )
    h1_ref[...] = h1
    p2_ref[...] = _dot(h1, wl2_ref[...])
    inv_ref[...] = inv


def _tc1(acc, accd, x, Wl1, Wr1, b1, Wl2):
    full = lambda i: (0, 0)
    return pl.pallas_call(
        _tc1_body,
        grid=(N_NODES // _R,),
        in_specs=[
            pl.BlockSpec((NC, _R, IN_CH), lambda i: (0, i, 0)),
            pl.BlockSpec((NC, _R, DEG_W), lambda i: (0, i, 0)),
            pl.BlockSpec((_R, IN_CH), lambda i: (i, 0)),
            pl.BlockSpec((IN_CH, HID), full),
            pl.BlockSpec((IN_CH, HID), full),
            pl.BlockSpec((1, HID), full),
            pl.BlockSpec((HID, 32), full),
        ],
        out_specs=[
            pl.BlockSpec((_R, HID), lambda i: (i, 0)),
            pl.BlockSpec((_R, 32), lambda i: (i, 0)),
            pl.BlockSpec((_R, 1), lambda i: (i, 0)),
        ],
        out_shape=[
            jax.ShapeDtypeStruct((N_NODES, HID), _F32),
            jax.ShapeDtypeStruct((N_NODES, 32), _F32),
            jax.ShapeDtypeStruct((N_NODES, 1), _F32),
        ],
    )(acc, accd, x, Wl1, Wr1, b1.reshape(1, HID), Wl2)


def _tc2_body(acc_ref, inv_ref, h1_ref, wr2_ref, b2_ref, wl3_ref,
              h2_ref, p3_ref):
    a = (acc_ref[0] + acc_ref[1]) * inv_ref[...]
    h2 = jnp.maximum(a + b2_ref[...] + _dot(h1_ref[...], wr2_ref[...]), 0.0)
    h2_ref[...] = h2
    p3_ref[...] = _dot(h2, wl3_ref[...])


def _tc2(acc, inv, h1, Wr2, b2, Wl3):
    full = lambda i: (0, 0)
    return pl.pallas_call(
        _tc2_body,
        grid=(N_NODES // _R,),
        in_specs=[
            pl.BlockSpec((NC, _R, 32), lambda i: (0, i, 0)),
            pl.BlockSpec((_R, 1), lambda i: (i, 0)),
            pl.BlockSpec((_R, HID), lambda i: (i, 0)),
            pl.BlockSpec((HID, 32), full),
            pl.BlockSpec((1, 32), full),
            pl.BlockSpec((32, 16), full),
        ],
        out_specs=[
            pl.BlockSpec((_R, 32), lambda i: (i, 0)),
            pl.BlockSpec((_R, 16), lambda i: (i, 0)),
        ],
        out_shape=[
            jax.ShapeDtypeStruct((N_NODES, 32), _F32),
            jax.ShapeDtypeStruct((N_NODES, 16), _F32),
        ],
    )(acc, inv, h1, Wr2, b2.reshape(1, 32), Wl3)


def _tc3_body(acc_ref, inv_ref, h2_ref, wr3_ref, b3_ref, fw1_ref, fb1_ref,
              fw2_ref, fb2_ref, out_ref):
    a = (acc_ref[0] + acc_ref[1]) * inv_ref[...]
    h3 = jnp.maximum(a + b3_ref[...] + _dot(h2_ref[...], wr3_ref[...]), 0.0)
    t = jnp.maximum(_dot(h3, fw1_ref[...]) + fb1_ref[...], 0.0)
    o = _dot(t, fw2_ref[...]) + fb2_ref[...]
    m = jnp.max(o, axis=1, keepdims=True)
    sh = o - m
    out_ref[...] = sh - jnp.log(jnp.sum(jnp.exp(sh), axis=1, keepdims=True))


def _tc3(acc, inv, h2, Wr3, b3, fcW1, fcb1, fcW2, fcb2):
    full = lambda i: (0, 0)
    return pl.pallas_call(
        _tc3_body,
        grid=(N_NODES // _R,),
        in_specs=[
            pl.BlockSpec((NC, _R, 16), lambda i: (0, i, 0)),
            pl.BlockSpec((_R, 1), lambda i: (i, 0)),
            pl.BlockSpec((_R, 32), lambda i: (i, 0)),
            pl.BlockSpec((32, 16), full),
            pl.BlockSpec((1, 16), full),
            pl.BlockSpec((16, 8), full),
            pl.BlockSpec((1, 8), full),
            pl.BlockSpec((8, 2), full),
            pl.BlockSpec((1, 2), full),
        ],
        out_specs=pl.BlockSpec((_R, 2), lambda i: (i, 0)),
        out_shape=jax.ShapeDtypeStruct((N_NODES, 2), _F32),
    )(acc, inv, h2, Wr3, b3.reshape(1, 16), fcW1, fcb1.reshape(1, 8),
      fcW2, fcb2.reshape(1, 2))


def kernel(x, edge_index, Wl1, Wr1, b1, Wl2, Wr2, b2, Wl3, Wr3, b3,
           fcW1, fcb1, fcW2, fcb2):
    edges = edge_index.astype(jnp.int32).reshape(2, NW, J, BLK)

    acc1, accd = _make_sc_agg(IN_CH, True)(
        x, edges, jnp.zeros((RPT, IN_CH), _F32),
        jnp.zeros((RPT, DEG_W), _F32), jnp.ones((BLK, DEG_W), _F32))
    h1, p2, inv = _tc1(acc1, accd, x, Wl1, Wr1, b1, Wl2)
    acc2, = _make_sc_agg(32, False)(p2, edges, jnp.zeros((RPT, 32), _F32))
    h2, p3 = _tc2(acc2, inv, h1, Wr2, b2, Wl3)
    acc3, = _make_sc_agg(16, False)(p3, edges, jnp.zeros((RPT, 16), _F32))
    return _tc3(acc3, inv, h2, Wr3, b3, fcW1, fcb1, fcW2, fcb2)


# speedup vs baseline: 17.7632x; 1.0497x over previous
import functools

import jax
import jax.numpy as jnp
from jax import lax
from jax.experimental import pallas as pl
from jax.experimental.pallas import tpu as pltpu
from jax.experimental.pallas import tpu_sc as plsc

N_NODES = 10000
N_EDGES = 320000
IN_CH = 128
HID = 256

NC = 2
NS = 16
NW = NC * NS
BLK = 40
J = N_EDGES // (NW * BLK)
ND = 4
ROWS_ACC = 10240
RPT = ROWS_ACC // NS
DEG_W = 8

_F32 = jnp.float32


@functools.cache
def _make_sc_agg(width, with_deg):
    mesh = plsc.VectorSubcoreMesh(core_axis_name="c", subcore_axis_name="s")

    out_type = [jax.ShapeDtypeStruct((NC, ROWS_ACC, width), _F32)]
    scratch = [
        pltpu.VMEM((J, BLK), jnp.int32),
        pltpu.VMEM((J, BLK), jnp.int32),
        pltpu.VMEM((ND, BLK, width), _F32),
        pltpu.MemorySpace.VMEM_SHARED((ROWS_ACC, width), _F32),
    ]
    scratch += [pltpu.SemaphoreType.DMA] * (2 * ND)
    if with_deg:
        out_type.append(jax.ShapeDtypeStruct((NC, ROWS_ACC, DEG_W), _F32))
        scratch += [
            pltpu.VMEM((BLK, DEG_W), _F32),
            pltpu.MemorySpace.VMEM_SHARED((ROWS_ACC, DEG_W), _F32),
            pltpu.SemaphoreType.DMA,
        ]

    @functools.partial(
        pl.kernel,
        out_type=out_type,
        mesh=mesh,
        scratch_types=scratch,
        compiler_params=pltpu.CompilerParams(use_tc_tiling_on_sc=False),
    )
    def sc_agg(table, edges, zrow, *rest):
        if with_deg:
            zdeg, ones, out, outd, src_v, dst_v, buf, acc = rest[:8]
            gsem = rest[8:8 + ND]
            ssem = rest[8 + ND:8 + 2 * ND]
            ones_v, accd, dsem = rest[8 + 2 * ND:]
        else:
            out, src_v, dst_v, buf, acc = rest[:5]
            gsem = rest[5:5 + ND]
            ssem = rest[5 + ND:5 + 2 * ND]
        c = lax.axis_index("c")
        s = lax.axis_index("s")
        wid = s * NC + c
        pltpu.sync_copy(edges.at[0, wid], src_v)
        pltpu.sync_copy(edges.at[1, wid], dst_v)

        def gather(j, slot):
            pltpu.async_copy(table.at[src_v.at[j]], buf.at[slot], gsem[slot])

        def wait_gather(j, slot):
            pltpu.make_async_copy(
                table.at[src_v.at[j]], buf.at[slot], gsem[slot]).wait()

        def scat(j, slot):
            pltpu.async_copy(
                buf.at[slot], acc.at[dst_v.at[j]], ssem[slot], add=True)

        def wait_scat(j, slot):
            pltpu.make_async_copy(
                buf.at[slot], acc.at[dst_v.at[j]], ssem[slot]).wait()

        for jp in range(ND - 1):
            gather(jp, jp)
        pltpu.sync_copy(zrow, acc.at[pl.ds(s * RPT, RPT)])
        if with_deg:
            pltpu.sync_copy(ones, ones_v)
            pltpu.sync_copy(zdeg, accd.at[pl.ds(s * RPT, RPT)])
        plsc.subcore_barrier()

        def do_j(j, u):
            up = (u + ND - 1) % ND
            wait_gather(j, u)
            scat(j, u)
            if with_deg:
                @pl.when(j >= 1)
                def _():
                    pltpu.make_async_copy(
                        ones_v, accd.at[dst_v.at[j - 1]], dsem).wait()

                pltpu.async_copy(
                    ones_v, accd.at[dst_v.at[j]], dsem, add=True)

            @pl.when(j + (ND - 1) < J)
            def _():
                @pl.when(j >= 1)
                def _():
                    wait_scat(j - 1, up)

                gather(j + ND - 1, up)

        def step(i, carry):
            for u in range(ND):
                do_j(i * ND + u, u)
            return carry

        n_main = (J // ND - 1) * ND
        lax.fori_loop(0, n_main // ND, step, 0)
        for j in range(n_main, J):
            do_j(j, j % ND)
        for j in range(J - ND, J):
            wait_scat(j, j % ND)
        if with_deg:
            pltpu.make_async_copy(
                ones_v, accd.at[dst_v.at[J - 1]], dsem).wait()
        plsc.subcore_barrier()
        pltpu.sync_copy(
            acc.at[pl.ds(s * RPT, RPT)], out.at[c, pl.ds(s * RPT, RPT)]
        )
        if with_deg:
            pltpu.sync_copy(
                accd.at[pl.ds(s * RPT, RPT)], outd.at[c, pl.ds(s * RPT, RPT)]
            )

    return sc_agg


def _dot(a, b):
    return jnp.dot(a, b, preferred_element_type=_F32)


_R = 2000


def _tc1_body(acc_ref, accd_ref, x_ref, wl1_ref, wr1_ref, b1_ref, wl2_ref,
              h1_ref, p2_ref, inv_ref):
    deg = accd_ref[0, :, :1] + accd_ref[1, :, :1]
    inv = 1.0 / jnp.maximum(deg, 1.0)
    aggn = (acc_ref[0] + acc_ref[1]) * inv
    h1 = jnp.maximum(
        _dot(aggn, wl1_ref[...]) + b1_ref[...] + _dot(x_ref[...], wr1_ref[...]),
        0.0,
    )
    h1_ref[...] = h1
    p2_ref[...] = _dot(h1, wl2_ref[...])
    inv_ref[...] = inv


def _tc1(acc, accd, x, Wl1, Wr1, b1, Wl2):
    full = lambda i: (0, 0)
    return pl.pallas_call(
        _tc1_body,
        grid=(N_NODES // _R,),
        in_specs=[
            pl.BlockSpec((NC, _R, IN_CH), lambda i: (0, i, 0)),
            pl.BlockSpec((NC, _R, DEG_W), lambda i: (0, i, 0)),
            pl.BlockSpec((_R, IN_CH), lambda i: (i, 0)),
            pl.BlockSpec((IN_CH, HID), full),
            pl.BlockSpec((IN_CH, HID), full),
            pl.BlockSpec((1, HID), full),
            pl.BlockSpec((HID, 32), full),
        ],
        out_specs=[
            pl.BlockSpec((_R, HID), lambda i: (i, 0)),
            pl.BlockSpec((_R, 32), lambda i: (i, 0)),
            pl.BlockSpec((_R, 1), lambda i: (i, 0)),
        ],
        out_shape=[
            jax.ShapeDtypeStruct((N_NODES, HID), _F32),
            jax.ShapeDtypeStruct((N_NODES, 32), _F32),
            jax.ShapeDtypeStruct((N_NODES, 1), _F32),
        ],
    )(acc, accd, x, Wl1, Wr1, b1.reshape(1, HID), Wl2)


def _tc2_body(acc_ref, inv_ref, h1_ref, wr2_ref, b2_ref, wl3_ref,
              h2_ref, p3_ref):
    a = (acc_ref[0] + acc_ref[1]) * inv_ref[...]
    h2 = jnp.maximum(a + b2_ref[...] + _dot(h1_ref[...], wr2_ref[...]), 0.0)
    h2_ref[...] = h2
    p3_ref[...] = _dot(h2, wl3_ref[...])


def _tc2(acc, inv, h1, Wr2, b2, Wl3):
    full = lambda i: (0, 0)
    return pl.pallas_call(
        _tc2_body,
        grid=(N_NODES // _R,),
        in_specs=[
            pl.BlockSpec((NC, _R, 32), lambda i: (0, i, 0)),
            pl.BlockSpec((_R, 1), lambda i: (i, 0)),
            pl.BlockSpec((_R, HID), lambda i: (i, 0)),
            pl.BlockSpec((HID, 32), full),
            pl.BlockSpec((1, 32), full),
            pl.BlockSpec((32, 16), full),
        ],
        out_specs=[
            pl.BlockSpec((_R, 32), lambda i: (i, 0)),
            pl.BlockSpec((_R, 16), lambda i: (i, 0)),
        ],
        out_shape=[
            jax.ShapeDtypeStruct((N_NODES, 32), _F32),
            jax.ShapeDtypeStruct((N_NODES, 16), _F32),
        ],
    )(acc, inv, h1, Wr2, b2.reshape(1, 32), Wl3)


def _tc3_body(acc_ref, inv_ref, h2_ref, wr3_ref, b3_ref, fw1_ref, fb1_ref,
              fw2_ref, fb2_ref, out_ref):
    a = (acc_ref[0] + acc_ref[1]) * inv_ref[...]
    h3 = jnp.maximum(a + b3_ref[...] + _dot(h2_ref[...], wr3_ref[...]), 0.0)
    t = jnp.maximum(_dot(h3, fw1_ref[...]) + fb1_ref[...], 0.0)
    o = _dot(t, fw2_ref[...]) + fb2_ref[...]
    m = jnp.max(o, axis=1, keepdims=True)
    sh = o - m
    out_ref[...] = sh - jnp.log(jnp.sum(jnp.exp(sh), axis=1, keepdims=True))


def _tc3(acc, inv, h2, Wr3, b3, fcW1, fcb1, fcW2, fcb2):
    full = lambda i: (0, 0)
    return pl.pallas_call(
        _tc3_body,
        grid=(N_NODES // _R,),
        in_specs=[
            pl.BlockSpec((NC, _R, 16), lambda i: (0, i, 0)),
            pl.BlockSpec((_R, 1), lambda i: (i, 0)),
            pl.BlockSpec((_R, 32), lambda i: (i, 0)),
            pl.BlockSpec((32, 16), full),
            pl.BlockSpec((1, 16), full),
            pl.BlockSpec((16, 8), full),
            pl.BlockSpec((1, 8), full),
            pl.BlockSpec((8, 2), full),
            pl.BlockSpec((1, 2), full),
        ],
        out_specs=pl.BlockSpec((_R, 2), lambda i: (i, 0)),
        out_shape=jax.ShapeDtypeStruct((N_NODES, 2), _F32),
    )(acc, inv, h2, Wr3, b3.reshape(1, 16), fcW1, fcb1.reshape(1, 8),
      fcW2, fcb2.reshape(1, 2))


def kernel(x, edge_index, Wl1, Wr1, b1, Wl2, Wr2, b2, Wl3, Wr3, b3,
           fcW1, fcb1, fcW2, fcb2):
    edges = edge_index.astype(jnp.int32).reshape(2, NW, J, BLK)

    acc1, accd = _make_sc_agg(IN_CH, True)(
        x, edges, jnp.zeros((RPT, IN_CH), _F32),
        jnp.zeros((RPT, DEG_W), _F32), jnp.ones((BLK, DEG_W), _F32))
    h1, p2, inv = _tc1(acc1, accd, x, Wl1, Wr1, b1, Wl2)
    acc2, = _make_sc_agg(32, False)(p2, edges, jnp.zeros((RPT, 32), _F32))
    h2, p3 = _tc2(acc2, inv, h1, Wr2, b2, Wl3)
    acc3, = _make_sc_agg(16, False)(p3, edges, jnp.zeros((RPT, 16), _F32))
    return _tc3(acc3, inv, h2, Wr3, b3, fcW1, fcb1, fcW2, fcb2)
